# baseline (speedup 1.0000x reference)
import functools

import jax
import jax.numpy as jnp
from jax import lax
from jax.experimental import pallas as pl
from jax.experimental.pallas import tpu as pltpu
from jax.experimental.pallas import tpu_sc as plsc

VOCAB = 1000000
DIM = 64
B = 16384
K = 20
SLOTS = K + 1

NC = 2
NS = 16
NW = NC * NS
BPW = B // NW
CB = 16
NCHUNK = BPW // CB
NSP = NCHUNK * SLOTS


def _rowsum16(tr):
    acc = tr[pl.ds(0, 16)]
    for l in range(1, 16):
        acc = acc + tr[pl.ds(l * 16, 16)]
    return acc


def _tile_slice(w, idx):
    t8 = (idx // 8) * 8
    return w.at[pl.ds(pl.multiple_of(t8, 8), 8)]


def _sg_body(cw, pw, nw, w_in, w_out, pos_out, neg_out,
             ti, pi, ni2, vin2, ctx2, po, no2, tr, sem_vin, sem_ctx):
    wid = lax.axis_index("s") * NC + lax.axis_index("c")
    wbase = wid * BPW
    lanes = lax.iota(jnp.int32, 16)

    pltpu.sync_copy(cw.at[pl.ds(wbase, BPW)], ti)
    pltpu.sync_copy(pw.at[pl.ds(wbase, BPW)], pi)
    pltpu.sync_copy(nw.at[pl.ds(wbase, CB)], ni2.at[0])

    def ctx_idx(c, j, pc, r):
        jm1 = lax.max(j - 1, 0)
        neg = ni2[pc, r, pl.ds(jm1, 16)][0]
        pos = pi[pl.ds(c * CB, 16)][r]
        return lax.select(j == 0, pos, neg)

    def issue_vin(c, p):
        tiv = ti[pl.ds(c * CB, 16)]
        for r in range(16):
            pltpu.async_copy(_tile_slice(w_in, tiv[r]), vin2.at[p, r],
                             sem_vin.at[p])

    def drain_vin(p):
        for r in range(16):
            pltpu.make_async_copy(w_in.at[pl.ds(0, 8)], vin2.at[p, r],
                                  sem_vin.at[p]).wait()

    def issue_ctx(sp):
        c = sp // SLOTS
        j = sp - c * SLOTS
        pc = lax.rem(c, 2)
        ps = lax.rem(sp, 2)
        for r in range(16):
            idx = ctx_idx(c, j, pc, r)
            pltpu.async_copy(_tile_slice(w_out, idx), ctx2.at[ps, r],
                             sem_ctx.at[ps])

    def drain_ctx(ps):
        for r in range(16):
            pltpu.make_async_copy(w_out.at[pl.ds(0, 8)], ctx2.at[ps, r],
                                  sem_ctx.at[ps]).wait()

    issue_vin(0, 0)
    issue_ctx(0)

    def sp_body(sp, _):
        c = sp // SLOTS
        j = sp - c * SLOTS
        pc = lax.rem(c, 2)
        ps = lax.rem(sp, 2)

        @pl.when(j == 0)
        def _():
            @pl.when(c + 1 < NCHUNK)
            def _():
                pltpu.sync_copy(nw.at[pl.ds(wbase + (c + 1) * CB, CB)],
                                ni2.at[lax.rem(c + 1, 2)])
                issue_vin(c + 1, lax.rem(c + 1, 2))

            drain_vin(pc)

        @pl.when(sp + 1 < NSP)
        def _():
            issue_ctx(sp + 1)

        drain_ctx(ps)

        tiv = ti[pl.ds(c * CB, 16)]
        for r in range(16):
            a_idx = tiv[r]
            b_idx = ctx_idx(c, j, pc, r)
            sa = a_idx - (a_idx // 8) * 8
            sb = b_idx - (b_idx // 8) * 8
            acc = vin2[pc, r, sa, pl.ds(0, 16)] * ctx2[ps, r, sb, pl.ds(0, 16)]
            for q in range(1, 4):
                acc = acc + (vin2[pc, r, sa, pl.ds(q * 16, 16)]
                             * ctx2[ps, r, sb, pl.ds(q * 16, 16)])
            plsc.store_scatter(tr, [lanes * 16 + r], acc)

        scores = _rowsum16(tr)

        @pl.when(j == 0)
        def _():
            po[pl.ds(c * CB, 16)] = scores

        @pl.when(j > 0)
        def _():
            plsc.store_scatter(no2.at[pc], [lanes, lanes - lanes + (j - 1)],
                               scores)

        @pl.when(j == SLOTS - 1)
        def _():
            pltpu.sync_copy(no2.at[pc],
                            neg_out.at[pl.ds(wbase + c * CB, CB)])

        return 0

    lax.fori_loop(0, NSP, sp_body, 0)
    pltpu.sync_copy(po, pos_out.at[pl.ds(wbase, BPW)])


_sg_call = functools.partial(
    pl.kernel,
    out_type=[
        jax.ShapeDtypeStruct((B,), jnp.float32),
        jax.ShapeDtypeStruct((B, K), jnp.float32),
    ],
    mesh=plsc.VectorSubcoreMesh(core_axis_name="c", subcore_axis_name="s"),
    compiler_params=pltpu.CompilerParams(
        needs_layout_passes=False,
        use_tc_tiling_on_sc=True,
        disable_bounds_checks=True,
    ),
    scratch_types=[
        pltpu.VMEM((BPW,), jnp.int32),
        pltpu.VMEM((BPW,), jnp.int32),
        pltpu.VMEM((2, CB, K), jnp.int32),
        pltpu.VMEM((2, 16, 8, DIM), jnp.float32),
        pltpu.VMEM((2, 16, 8, DIM), jnp.float32),
        pltpu.VMEM((BPW,), jnp.float32),
        pltpu.VMEM((2, CB, K), jnp.float32),
        pltpu.VMEM((256,), jnp.float32),
        pltpu.SemaphoreType.DMA((2,)),
        pltpu.SemaphoreType.DMA((2,)),
    ],
)(_sg_body)


def kernel(center_words, pos_context_words, neg_context_words, W_in, W_out):
    cw = center_words.astype(jnp.int32)
    pw = pos_context_words.astype(jnp.int32)
    nw = neg_context_words.astype(jnp.int32)
    pos_scores, neg_scores = _sg_call(cw, pw, nw, W_in, W_out)
    return pos_scores, neg_scores
